# baseline (speedup 1.0000x reference)
import jax
import jax.numpy as jnp
from jax import lax
from jax.experimental import pallas as pl
from jax.experimental.pallas import tpu as pltpu
from jax.experimental.pallas import tpu_sc as plsc

_R = 512
_C = 1024

_NC = 2
_NS = 16
_NW = _NC * _NS
_CHUNK = 32768
_SC_ELEMS = 12 * 1024 * 1024


def _minmax_body(a_ref, out_ref):
    i = pl.program_id(0)
    a = a_ref[...]
    amin, amax = jnp.min(a), jnp.max(a)

    @pl.when(i == 0)
    def _init():
        out_ref[0] = amin
        out_ref[1] = amax

    @pl.when(i != 0)
    def _acc():
        out_ref[0] = jnp.minimum(out_ref[0], amin)
        out_ref[1] = jnp.maximum(out_ref[1], amax)


def _tc_hist_body(ed_ref, x_ref, out_ref, acc_ref):
    i = pl.program_id(0)

    @pl.when(i == 0)
    def _init():
        for j in range(9):
            acc_ref[j] = 0.0

    x = x_ref[...]
    for j in range(9):
        acc_ref[j] += jnp.sum((x >= ed_ref[j]).astype(jnp.float32))

    @pl.when(i == pl.num_programs(0) - 1)
    def _fin():
        for j in range(9):
            out_ref[j] = acc_ref[j]
        for j in range(9, 16):
            out_ref[j] = 0.0


def _sc_minmax_body(x_hbm, out_hbm, buf0, buf1, stage_v, sem0, sem1):
    c = lax.axis_index("c")
    s = lax.axis_index("s")
    wid = s * _NC + c
    perw = x_hbm.shape[0] // _NW
    base = wid * perw
    npairs = perw // (2 * _CHUNK)

    def _dma_start(ci, buf, sem):
        pltpu.async_copy(x_hbm.at[pl.ds(base + ci * _CHUNK, _CHUNK)], buf, sem)

    def _dma_wait(buf, sem):
        pltpu.make_async_copy(x_hbm.at[pl.ds(base, _CHUNK)], buf, sem).wait()

    def _process(buf, mm):
        @plsc.parallel_loop(0, _CHUNK // 16, 1, unroll=8, carry=mm)
        def _it(i, mm):
            v = buf[pl.ds(i * 16, 16)]
            return jnp.minimum(mm[0], v), jnp.maximum(mm[1], v)
        return _it

    _dma_start(0, buf0, sem0)
    v0 = jnp.full((16,), jnp.inf, jnp.float32)
    v1 = jnp.full((16,), -jnp.inf, jnp.float32)

    def _pair(p, mm):
        c0 = p * 2
        _dma_start(c0 + 1, buf1, sem1)
        _dma_wait(buf0, sem0)
        mm = _process(buf0, mm)

        @pl.when(p < npairs - 1)
        def _():
            _dma_start(c0 + 2, buf0, sem0)

        _dma_wait(buf1, sem1)
        mm = _process(buf1, mm)
        return mm

    vmin, vmax = lax.fori_loop(0, npairs, _pair, (v0, v1))
    stage_v[0] = vmin
    stage_v[1] = vmax
    pltpu.sync_copy(stage_v, out_hbm.at[wid])


_sc_minmax = pl.kernel(
    _sc_minmax_body,
    out_type=jax.ShapeDtypeStruct((_NW, 2, 16), jnp.float32),
    mesh=plsc.VectorSubcoreMesh(
        core_axis_name="c", subcore_axis_name="s",
        num_cores=_NC, num_subcores=_NS),
    compiler_params=pltpu.CompilerParams(needs_layout_passes=False),
    scratch_types=[
        pltpu.VMEM((_CHUNK,), jnp.float32),
        pltpu.VMEM((_CHUNK,), jnp.float32),
        pltpu.VMEM((2, 16), jnp.float32),
        pltpu.SemaphoreType.DMA,
        pltpu.SemaphoreType.DMA,
    ],
)


def _sc_hist_body(x_hbm, par_hbm, etab_hbm, out_hbm,
                  buf0, buf1, par_v, etab_v, tbl_v, sem0, sem1):
    c = lax.axis_index("c")
    s = lax.axis_index("s")
    wid = s * _NC + c
    perw = _SC_ELEMS // _NW
    base = wid * perw
    npairs = perw // (2 * _CHUNK)

    pltpu.sync_copy(par_hbm, par_v)
    pltpu.sync_copy(etab_hbm, etab_v)
    for j in range(16):
        tbl_v[j] = jnp.zeros((16,), jnp.float32)

    lo_v = par_v[0]
    scale_v = par_v[1]
    bias_v = par_v[2]
    lane = lax.iota(jnp.int32, 16)
    nine = jnp.full((16,), 9, jnp.int32)
    one_i = jnp.full((16,), 1, jnp.int32)
    zero_i = jnp.full((16,), 0, jnp.int32)
    ones_f = jnp.ones((16,), jnp.float32)

    def _dma_start(ci, buf, sem):
        pltpu.async_copy(x_hbm.at[pl.ds(base + ci * _CHUNK, _CHUNK)], buf, sem)

    def _dma_wait(buf, sem):
        pltpu.make_async_copy(x_hbm.at[pl.ds(base, _CHUNK)], buf, sem).wait()

    def _process(buf):
        @plsc.parallel_loop(0, _CHUNK // 16, 1, unroll=8)
        def _it(i):
            v = buf[pl.ds(i * 16, 16)]
            t = (v - lo_v) * scale_v + bias_v
            i0 = jnp.minimum(t.astype(jnp.int32), nine)
            e_hi = plsc.load_gather(etab_v, [i0 + one_i])
            e_lo = plsc.load_gather(etab_v, [i0])
            cc = (i0 + jnp.where(v >= e_hi, one_i, zero_i)
                  - jnp.where(v < e_lo, one_i, zero_i))
            plsc.addupdate_scatter(tbl_v, [cc, lane], ones_f)

    _dma_start(0, buf0, sem0)

    def _pair(p, carry):
        c0 = p * 2
        _dma_start(c0 + 1, buf1, sem1)
        _dma_wait(buf0, sem0)
        _process(buf0)

        @pl.when(p < npairs - 1)
        def _():
            _dma_start(c0 + 2, buf0, sem0)

        _dma_wait(buf1, sem1)
        _process(buf1)
        return carry

    lax.fori_loop(0, npairs, _pair, 0)
    pltpu.sync_copy(tbl_v, out_hbm.at[wid])


_sc_hist = pl.kernel(
    _sc_hist_body,
    out_type=jax.ShapeDtypeStruct((_NW, 16, 16), jnp.float32),
    mesh=plsc.VectorSubcoreMesh(
        core_axis_name="c", subcore_axis_name="s",
        num_cores=_NC, num_subcores=_NS),
    compiler_params=pltpu.CompilerParams(needs_layout_passes=False),
    scratch_types=[
        pltpu.VMEM((_CHUNK,), jnp.float32),
        pltpu.VMEM((_CHUNK,), jnp.float32),
        pltpu.VMEM((4, 16), jnp.float32),
        pltpu.VMEM((16,), jnp.float32),
        pltpu.VMEM((16, 16), jnp.float32),
        pltpu.SemaphoreType.DMA,
        pltpu.SemaphoreType.DMA,
    ],
)


def _sc_histogram(x, lo, hi):
    degen = hi <= lo
    edges = jnp.linspace(lo, hi, 11)
    scale = jnp.where(degen, jnp.float32(0.0), jnp.float32(10.0) / (hi - lo))
    bias = jnp.where(degen, jnp.float32(9.0), jnp.float32(0.0))
    etab = jnp.concatenate([edges, jnp.zeros((5,), jnp.float32)])
    etab = etab.at[10].set(jnp.inf)
    degen_tab = jnp.concatenate([
        jnp.full((10,), -jnp.inf, jnp.float32),
        jnp.full((6,), jnp.inf, jnp.float32),
    ])
    etab = jnp.where(degen, degen_tab, etab)
    params = jnp.stack([
        jnp.full((16,), lo, jnp.float32),
        jnp.full((16,), scale, jnp.float32),
        jnp.full((16,), bias, jnp.float32),
        jnp.zeros((16,), jnp.float32),
    ])
    parts = _sc_hist(x, params, etab)
    return parts.sum(axis=(0, 2))[:10]


def kernel(opt_tensor, tensor):
    n = opt_tensor.shape[0]
    a2 = opt_tensor.reshape(n // _C, _C)
    nb = n // (_R * _C)

    mm_t = _sc_minmax(tensor)
    tmin = jnp.min(mm_t[:, 0, :])
    tmax = jnp.max(mm_t[:, 1, :])

    mm = pl.pallas_call(
        _minmax_body,
        grid=(nb,),
        in_specs=[pl.BlockSpec((_R, _C), lambda i: (i, 0))],
        out_specs=pl.BlockSpec(memory_space=pltpu.SMEM),
        out_shape=jax.ShapeDtypeStruct((2,), jnp.float32),
    )(a2)

    edges_o = jnp.linspace(mm[0], mm[1], 11)

    d = pl.pallas_call(
        _tc_hist_body,
        grid=(nb,),
        in_specs=[
            pl.BlockSpec(memory_space=pltpu.SMEM),
            pl.BlockSpec((_R, _C), lambda i: (i, 0)),
        ],
        out_specs=pl.BlockSpec(memory_space=pltpu.SMEM),
        out_shape=jax.ShapeDtypeStruct((16,), jnp.float32),
        scratch_shapes=[pltpu.SMEM((16,), jnp.float32)],
    )(edges_o[1:10], a2)

    b2 = tensor.reshape(n // _C, _C)
    tail_blocks = (n - _SC_ELEMS) // (_R * _C)
    off = _SC_ELEMS // (_R * _C)
    edges_t = jnp.linspace(tmin, tmax, 11)
    dt = pl.pallas_call(
        _tc_hist_body,
        grid=(tail_blocks,),
        in_specs=[
            pl.BlockSpec(memory_space=pltpu.SMEM),
            pl.BlockSpec((_R, _C), lambda i: (i + off, 0)),
        ],
        out_specs=pl.BlockSpec(memory_space=pltpu.SMEM),
        out_shape=jax.ShapeDtypeStruct((16,), jnp.float32),
        scratch_shapes=[pltpu.SMEM((16,), jnp.float32)],
    )(edges_t[1:10], b2)

    nf = jnp.float32(n)
    ntail = jnp.float32(n - _SC_ELEMS)
    counts_o = jnp.concatenate(
        [nf - d[:1], d[:8] - d[1:9], d[8:9]])
    counts_tail = jnp.concatenate(
        [ntail - dt[:1], dt[:8] - dt[1:9], dt[8:9]])
    counts_t = _sc_histogram(tensor, tmin, tmax) + counts_tail
    return jnp.mean((counts_o - counts_t) ** 2)


# speedup vs baseline: 1.4216x; 1.4216x over previous
import jax
import jax.numpy as jnp
from jax import lax
from jax.experimental import pallas as pl
from jax.experimental.pallas import tpu as pltpu
from jax.experimental.pallas import tpu_sc as plsc

_R = 512
_C = 1024

_NC = 2
_NS = 16
_NW = _NC * _NS
_CHUNK = 32768
_SC_ELEMS = 16 * 1024 * 1024


def _minmax_body(a_ref, out_ref):
    i = pl.program_id(0)
    a = a_ref[...]
    amin, amax = jnp.min(a), jnp.max(a)

    @pl.when(i == 0)
    def _init():
        out_ref[0] = amin
        out_ref[1] = amax

    @pl.when(i != 0)
    def _acc():
        out_ref[0] = jnp.minimum(out_ref[0], amin)
        out_ref[1] = jnp.maximum(out_ref[1], amax)


def _tc_hist_body(ed_ref, x_ref, out_ref, acc_ref):
    i = pl.program_id(0)

    @pl.when(i == 0)
    def _init():
        for j in range(9):
            acc_ref[j] = 0.0

    x = x_ref[...]
    for j in range(9):
        acc_ref[j] += jnp.sum((x >= ed_ref[j]).astype(jnp.float32))

    @pl.when(i == pl.num_programs(0) - 1)
    def _fin():
        for j in range(9):
            out_ref[j] = acc_ref[j]
        for j in range(9, 16):
            out_ref[j] = 0.0


def _sc_minmax2_body(a_hbm, b_hbm, out_hbm, buf0, buf1, stage_v, sem0, sem1):
    c = lax.axis_index("c")
    s = lax.axis_index("s")
    wid = s * _NC + c

    def _reduce_one(x_hbm, row):
        perw = x_hbm.shape[0] // _NW
        base = wid * perw
        npairs = perw // (2 * _CHUNK)

        def _dma_start(ci, buf, sem):
            pltpu.async_copy(
                x_hbm.at[pl.ds(base + ci * _CHUNK, _CHUNK)], buf, sem)

        def _dma_wait(buf, sem):
            pltpu.make_async_copy(
                x_hbm.at[pl.ds(base, _CHUNK)], buf, sem).wait()

        def _process(buf, mm):
            @plsc.parallel_loop(0, _CHUNK // 16, 1, unroll=8, carry=mm)
            def _it(i, mm):
                v = buf[pl.ds(i * 16, 16)]
                return jnp.minimum(mm[0], v), jnp.maximum(mm[1], v)
            return _it

        _dma_start(0, buf0, sem0)
        v0 = jnp.full((16,), jnp.inf, jnp.float32)
        v1 = jnp.full((16,), -jnp.inf, jnp.float32)

        def _pair(p, mm):
            c0 = p * 2
            _dma_start(c0 + 1, buf1, sem1)
            _dma_wait(buf0, sem0)
            mm = _process(buf0, mm)

            @pl.when(p < npairs - 1)
            def _():
                _dma_start(c0 + 2, buf0, sem0)

            _dma_wait(buf1, sem1)
            mm = _process(buf1, mm)
            return mm

        vmin, vmax = lax.fori_loop(0, npairs, _pair, (v0, v1))
        stage_v[row] = vmin
        stage_v[row + 1] = vmax

    _reduce_one(a_hbm, 0)
    _reduce_one(b_hbm, 2)
    pltpu.sync_copy(stage_v, out_hbm.at[wid])


_sc_minmax2 = pl.kernel(
    _sc_minmax2_body,
    out_type=jax.ShapeDtypeStruct((_NW, 4, 16), jnp.float32),
    mesh=plsc.VectorSubcoreMesh(
        core_axis_name="c", subcore_axis_name="s",
        num_cores=_NC, num_subcores=_NS),
    compiler_params=pltpu.CompilerParams(needs_layout_passes=False),
    scratch_types=[
        pltpu.VMEM((_CHUNK,), jnp.float32),
        pltpu.VMEM((_CHUNK,), jnp.float32),
        pltpu.VMEM((4, 16), jnp.float32),
        pltpu.SemaphoreType.DMA,
        pltpu.SemaphoreType.DMA,
    ],
)


def _sc_hist_body(x_hbm, par_hbm, etab_hbm, out_hbm,
                  buf0, buf1, par_v, etab_v, tbl_v, sem0, sem1):
    c = lax.axis_index("c")
    s = lax.axis_index("s")
    wid = s * _NC + c
    perw = _SC_ELEMS // _NW
    base = wid * perw
    npairs = perw // (2 * _CHUNK)

    pltpu.sync_copy(par_hbm, par_v)
    pltpu.sync_copy(etab_hbm, etab_v)
    for j in range(16):
        tbl_v[j] = jnp.zeros((16,), jnp.float32)

    lo_v = par_v[0]
    scale_v = par_v[1]
    bias_v = par_v[2]
    lane = lax.iota(jnp.int32, 16)
    nine = jnp.full((16,), 9, jnp.int32)
    one_i = jnp.full((16,), 1, jnp.int32)
    zero_i = jnp.full((16,), 0, jnp.int32)
    ones_f = jnp.ones((16,), jnp.float32)

    def _dma_start(ci, buf, sem):
        pltpu.async_copy(x_hbm.at[pl.ds(base + ci * _CHUNK, _CHUNK)], buf, sem)

    def _dma_wait(buf, sem):
        pltpu.make_async_copy(x_hbm.at[pl.ds(base, _CHUNK)], buf, sem).wait()

    def _process(buf):
        @plsc.parallel_loop(0, _CHUNK // 16, 1, unroll=8)
        def _it(i):
            v = buf[pl.ds(i * 16, 16)]
            t = (v - lo_v) * scale_v + bias_v
            i0 = jnp.minimum(t.astype(jnp.int32), nine)
            e_hi = plsc.load_gather(etab_v, [i0 + one_i])
            e_lo = plsc.load_gather(etab_v, [i0])
            cc = (i0 + jnp.where(v >= e_hi, one_i, zero_i)
                  - jnp.where(v < e_lo, one_i, zero_i))
            plsc.addupdate_scatter(tbl_v, [cc, lane], ones_f)

    _dma_start(0, buf0, sem0)

    def _pair(p, carry):
        c0 = p * 2
        _dma_start(c0 + 1, buf1, sem1)
        _dma_wait(buf0, sem0)
        _process(buf0)

        @pl.when(p < npairs - 1)
        def _():
            _dma_start(c0 + 2, buf0, sem0)

        _dma_wait(buf1, sem1)
        _process(buf1)
        return carry

    lax.fori_loop(0, npairs, _pair, 0)
    pltpu.sync_copy(tbl_v, out_hbm.at[wid])


_sc_hist = pl.kernel(
    _sc_hist_body,
    out_type=jax.ShapeDtypeStruct((_NW, 16, 16), jnp.float32),
    mesh=plsc.VectorSubcoreMesh(
        core_axis_name="c", subcore_axis_name="s",
        num_cores=_NC, num_subcores=_NS),
    compiler_params=pltpu.CompilerParams(needs_layout_passes=False),
    scratch_types=[
        pltpu.VMEM((_CHUNK,), jnp.float32),
        pltpu.VMEM((_CHUNK,), jnp.float32),
        pltpu.VMEM((4, 16), jnp.float32),
        pltpu.VMEM((16,), jnp.float32),
        pltpu.VMEM((16, 16), jnp.float32),
        pltpu.SemaphoreType.DMA,
        pltpu.SemaphoreType.DMA,
    ],
)


def _sc_histogram(x, lo, hi):
    degen = hi <= lo
    edges = jnp.linspace(lo, hi, 11)
    scale = jnp.where(degen, jnp.float32(0.0), jnp.float32(10.0) / (hi - lo))
    bias = jnp.where(degen, jnp.float32(9.0), jnp.float32(0.0))
    etab = jnp.concatenate([edges, jnp.zeros((5,), jnp.float32)])
    etab = etab.at[10].set(jnp.inf)
    degen_tab = jnp.concatenate([
        jnp.full((10,), -jnp.inf, jnp.float32),
        jnp.full((6,), jnp.inf, jnp.float32),
    ])
    etab = jnp.where(degen, degen_tab, etab)
    params = jnp.stack([
        jnp.full((16,), lo, jnp.float32),
        jnp.full((16,), scale, jnp.float32),
        jnp.full((16,), bias, jnp.float32),
        jnp.zeros((16,), jnp.float32),
    ])
    parts = _sc_hist(x, params, etab)
    return parts.sum(axis=(0, 2))[:10]


def kernel(opt_tensor, tensor):
    n = opt_tensor.shape[0]
    a2 = opt_tensor.reshape(n // _C, _C)
    nb = n // (_R * _C)

    mm = _sc_minmax2(opt_tensor, tensor)
    omin = jnp.min(mm[:, 0, :])
    omax = jnp.max(mm[:, 1, :])
    tmin = jnp.min(mm[:, 2, :])
    tmax = jnp.max(mm[:, 3, :])

    edges_o = jnp.linspace(omin, omax, 11)

    d = pl.pallas_call(
        _tc_hist_body,
        grid=(nb,),
        in_specs=[
            pl.BlockSpec(memory_space=pltpu.SMEM),
            pl.BlockSpec((_R, _C), lambda i: (i, 0)),
        ],
        out_specs=pl.BlockSpec(memory_space=pltpu.SMEM),
        out_shape=jax.ShapeDtypeStruct((16,), jnp.float32),
        scratch_shapes=[pltpu.SMEM((16,), jnp.float32)],
    )(edges_o[1:10], a2)

    nf = jnp.float32(n)
    counts_o = jnp.concatenate(
        [nf - d[:1], d[:8] - d[1:9], d[8:9]])
    counts_t = _sc_histogram(tensor, tmin, tmax)
    return jnp.mean((counts_o - counts_t) ** 2)
